# Initial kernel scaffold; baseline (speedup 1.0000x reference)
import jax
import jax.numpy as jnp
from jax.experimental import pallas as pl


def kernel(x, y, edge_index, W, gamma, beta):
    raise NotImplementedError("write your pallas kernel here")


# speedup vs baseline: 1.7282x; 1.7282x over previous
import functools

import jax
import jax.numpy as jnp
from jax import lax
from jax.experimental import pallas as pl
from jax.experimental.pallas import tpu as pltpu
from jax.experimental.pallas import tpu_sc as plsc

B, C, N, K = 2, 128, 10000, 16
O = 128
BN = B * N
NC, NS = 2, 16
NW = NC * NS
ROWS_PER_W = BN // NW
CHUNK = 5
NCHUNK = ROWS_PER_W // CHUNK
IDX_PER_CHUNK = CHUNK * K
LANES = 16

RB = 2000
GRID_A = BN // RB


def _sc_maxrel(xt, yt, e1, e0, out, idx1_v, idx0_v, gx, gy, ob, sem0, sem1):
    w = lax.axis_index("s") * NC + lax.axis_index("c")
    base_row = w * ROWS_PER_W

    def chunk_body(j, carry):
        r0 = base_row + j * CHUNK
        pltpu.sync_copy(e0.at[pl.ds(r0 * K, IDX_PER_CHUNK)], idx0_v)
        pltpu.sync_copy(e1.at[pl.ds(r0 * K, IDX_PER_CHUNK)], idx1_v)
        cy = pltpu.async_copy(yt.at[idx0_v], gy, sem0)
        cx = pltpu.async_copy(xt.at[idx1_v], gx, sem1)
        cy.wait()
        cx.wait()
        for r in range(CHUNK):
            for cc in range(C // LANES):
                cs = pl.ds(cc * LANES, LANES)
                m = gy[r * K, cs] - gx[r * K, cs]
                for k in range(1, K):
                    m = jnp.maximum(m, gy[r * K + k, cs] - gx[r * K + k, cs])
                ob[pl.ds(r * C + cc * LANES, LANES)] = m
        pltpu.sync_copy(ob, out.at[pl.ds(r0 * C, CHUNK * C)])
        return carry

    lax.fori_loop(0, NCHUNK, chunk_body, 0)


_sc_call = pl.kernel(
    _sc_maxrel,
    out_type=jax.ShapeDtypeStruct((BN * C,), jnp.float32),
    mesh=plsc.VectorSubcoreMesh(core_axis_name="c", subcore_axis_name="s"),
    scratch_types=[
        pltpu.VMEM((IDX_PER_CHUNK,), jnp.int32),
        pltpu.VMEM((IDX_PER_CHUNK,), jnp.int32),
        pltpu.VMEM((IDX_PER_CHUNK, C), jnp.float32),
        pltpu.VMEM((IDX_PER_CHUNK, C), jnp.float32),
        pltpu.VMEM((CHUNK * C,), jnp.float32),
        pltpu.SemaphoreType.DMA,
        pltpu.SemaphoreType.DMA,
    ],
)


def _tc_conv(xt_ref, mr_ref, w_ref, out_ref, sum_ref, sq_ref):
    i = pl.program_id(0)
    o = lax.dot_general(xt_ref[...], w_ref[:, :C], (((1,), (1,)), ((), ())),
                        preferred_element_type=jnp.float32)
    o += lax.dot_general(mr_ref[...], w_ref[:, C:], (((1,), (1,)), ((), ())),
                         preferred_element_type=jnp.float32)
    out_ref[...] = o

    @pl.when(i == 0)
    def _():
        sum_ref[...] = jnp.zeros_like(sum_ref)
        sq_ref[...] = jnp.zeros_like(sq_ref)

    sum_ref[...] += jnp.sum(o, axis=0, keepdims=True)
    sq_ref[...] += jnp.sum(o * o, axis=0, keepdims=True)


_tc_a = pl.pallas_call(
    _tc_conv,
    grid=(GRID_A,),
    in_specs=[
        pl.BlockSpec((RB, C), lambda i: (i, 0)),
        pl.BlockSpec((RB, C), lambda i: (i, 0)),
        pl.BlockSpec((O, 2 * C), lambda i: (0, 0)),
    ],
    out_specs=[
        pl.BlockSpec((RB, O), lambda i: (i, 0)),
        pl.BlockSpec((1, O), lambda i: (0, 0)),
        pl.BlockSpec((1, O), lambda i: (0, 0)),
    ],
    out_shape=[
        jax.ShapeDtypeStruct((BN, O), jnp.float32),
        jax.ShapeDtypeStruct((1, O), jnp.float32),
        jax.ShapeDtypeStruct((1, O), jnp.float32),
    ],
)


def _tc_bngelu(or_ref, sum_ref, sq_ref, g_ref, b_ref, out_ref):
    mean = sum_ref[...] * (1.0 / BN)
    var = sq_ref[...] * (1.0 / BN) - mean * mean
    scale = g_ref[...] * lax.rsqrt(var + 1e-5)
    shift = b_ref[...] - mean * scale
    out_ref[...] = jax.nn.gelu(or_ref[...] * scale + shift)


_tc_b = pl.pallas_call(
    _tc_bngelu,
    grid=(GRID_A,),
    in_specs=[
        pl.BlockSpec((RB, O), lambda i: (i, 0)),
        pl.BlockSpec((1, O), lambda i: (0, 0)),
        pl.BlockSpec((1, O), lambda i: (0, 0)),
        pl.BlockSpec((1, O), lambda i: (0, 0)),
        pl.BlockSpec((1, O), lambda i: (0, 0)),
    ],
    out_specs=pl.BlockSpec((RB, O), lambda i: (i, 0)),
    out_shape=jax.ShapeDtypeStruct((BN, O), jnp.float32),
)


def kernel(x, y, edge_index, W, gamma, beta):
    xt = x[:, :, :, 0].transpose(0, 2, 1).reshape(BN, C)
    yt = y[:, :, :, 0].transpose(0, 2, 1).reshape(BN, C)
    off = (jnp.arange(B, dtype=jnp.int32) * N).reshape(B, 1, 1)
    e0f = (edge_index[0] + off).reshape(-1)
    e1f = (edge_index[1] + off).reshape(-1)
    mr = _sc_call(xt, yt, e1f, e0f).reshape(BN, C)
    out_raw, s, q = _tc_a(xt, mr, W)
    out = _tc_b(out_raw, s, q, gamma.reshape(1, O), beta.reshape(1, O))
    return out.reshape(B, N, O).transpose(0, 2, 1)[..., None]


# speedup vs baseline: 2.4635x; 1.4254x over previous
import functools

import jax
import jax.numpy as jnp
from jax import lax
from jax.experimental import pallas as pl
from jax.experimental.pallas import tpu as pltpu
from jax.experimental.pallas import tpu_sc as plsc

B, C, N, K = 2, 128, 10000, 16
O = 128
BN = B * N
NC, NS = 2, 16
NW = NC * NS
ROWS_PER_W = BN // NW
CHUNK = 5
NCHUNK = ROWS_PER_W // CHUNK
IDX_PER_CHUNK = CHUNK * K
LANES = 16

RB = 2000
GRID_A = BN // RB


IDXW = ROWS_PER_W * K


def _sc_maxrel(xt, yt, e1, e0, out, ix_all, iy_all,
               gx0, gy0, gx1, gy1, ob0, ob1,
               sx0, sy0, sx1, sy1, so0, so1):
    w = lax.axis_index("s") * NC + lax.axis_index("c")
    base_row = w * ROWS_PER_W
    pltpu.sync_copy(e0.at[pl.ds(base_row * K, IDXW)], iy_all)
    pltpu.sync_copy(e1.at[pl.ds(base_row * K, IDXW)], ix_all)

    def start(j, gx, gy, semx, semy):
        isl = pl.ds(j * IDX_PER_CHUNK, IDX_PER_CHUNK)
        pltpu.async_copy(yt.at[iy_all.at[isl]], gy, semy)
        pltpu.async_copy(xt.at[ix_all.at[isl]], gx, semx)

    def wait(gx, gy, semx, semy):
        pltpu.make_async_copy(yt.at[iy_all.at[pl.ds(0, IDX_PER_CHUNK)]], gy, semy).wait()
        pltpu.make_async_copy(xt.at[ix_all.at[pl.ds(0, IDX_PER_CHUNK)]], gx, semx).wait()

    def compute(j, gx, gy, ob, semo, first):
        @pl.when(jnp.logical_not(first))
        def _():
            pltpu.make_async_copy(ob, out.at[pl.ds(0, CHUNK * C)], semo).wait()
        for r in range(CHUNK):
            for cc in range(C // LANES):
                cs = pl.ds(cc * LANES, LANES)
                m = gy[r * K, cs] - gx[r * K, cs]
                for k in range(1, K):
                    m = jnp.maximum(m, gy[r * K + k, cs] - gx[r * K + k, cs])
                ob[pl.ds(r * C + cc * LANES, LANES)] = m
        pltpu.async_copy(ob, out.at[pl.ds((base_row + j * CHUNK) * C, CHUNK * C)], semo)

    start(0, gx0, gy0, sx0, sy0)

    def pair_body(p, carry):
        j0 = 2 * p
        start(j0 + 1, gx1, gy1, sx1, sy1)
        wait(gx0, gy0, sx0, sy0)
        compute(j0, gx0, gy0, ob0, so0, p == 0)

        @pl.when(j0 + 2 < NCHUNK)
        def _():
            start(j0 + 2, gx0, gy0, sx0, sy0)
        wait(gx1, gy1, sx1, sy1)
        compute(j0 + 1, gx1, gy1, ob1, so1, p == 0)
        return carry

    lax.fori_loop(0, NCHUNK // 2, pair_body, 0)
    wait(gx0, gy0, sx0, sy0)
    compute(NCHUNK - 1, gx0, gy0, ob0, so0, False)
    pltpu.make_async_copy(ob0, out.at[pl.ds(0, CHUNK * C)], so0).wait()
    pltpu.make_async_copy(ob1, out.at[pl.ds(0, CHUNK * C)], so1).wait()


_sc_call = pl.kernel(
    _sc_maxrel,
    out_type=jax.ShapeDtypeStruct((BN * C,), jnp.float32),
    mesh=plsc.VectorSubcoreMesh(core_axis_name="c", subcore_axis_name="s"),
    scratch_types=[
        pltpu.VMEM((IDXW,), jnp.int32),
        pltpu.VMEM((IDXW,), jnp.int32),
        pltpu.VMEM((IDX_PER_CHUNK, C), jnp.float32),
        pltpu.VMEM((IDX_PER_CHUNK, C), jnp.float32),
        pltpu.VMEM((IDX_PER_CHUNK, C), jnp.float32),
        pltpu.VMEM((IDX_PER_CHUNK, C), jnp.float32),
        pltpu.VMEM((CHUNK * C,), jnp.float32),
        pltpu.VMEM((CHUNK * C,), jnp.float32),
        pltpu.SemaphoreType.DMA,
        pltpu.SemaphoreType.DMA,
        pltpu.SemaphoreType.DMA,
        pltpu.SemaphoreType.DMA,
        pltpu.SemaphoreType.DMA,
        pltpu.SemaphoreType.DMA,
    ],
)


def _tc_conv(xt_ref, mr_ref, w_ref, out_ref, sum_ref, sq_ref):
    i = pl.program_id(0)
    o = lax.dot_general(xt_ref[...], w_ref[:, :C], (((1,), (1,)), ((), ())),
                        preferred_element_type=jnp.float32)
    o += lax.dot_general(mr_ref[...], w_ref[:, C:], (((1,), (1,)), ((), ())),
                         preferred_element_type=jnp.float32)
    out_ref[...] = o

    @pl.when(i == 0)
    def _():
        sum_ref[...] = jnp.zeros_like(sum_ref)
        sq_ref[...] = jnp.zeros_like(sq_ref)

    sum_ref[...] += jnp.sum(o, axis=0, keepdims=True)
    sq_ref[...] += jnp.sum(o * o, axis=0, keepdims=True)


_tc_a = pl.pallas_call(
    _tc_conv,
    grid=(GRID_A,),
    in_specs=[
        pl.BlockSpec((RB, C), lambda i: (i, 0)),
        pl.BlockSpec((RB, C), lambda i: (i, 0)),
        pl.BlockSpec((O, 2 * C), lambda i: (0, 0)),
    ],
    out_specs=[
        pl.BlockSpec((RB, O), lambda i: (i, 0)),
        pl.BlockSpec((1, O), lambda i: (0, 0)),
        pl.BlockSpec((1, O), lambda i: (0, 0)),
    ],
    out_shape=[
        jax.ShapeDtypeStruct((BN, O), jnp.float32),
        jax.ShapeDtypeStruct((1, O), jnp.float32),
        jax.ShapeDtypeStruct((1, O), jnp.float32),
    ],
)


def _tc_bngelu(or_ref, sum_ref, sq_ref, g_ref, b_ref, out_ref):
    mean = sum_ref[...] * (1.0 / BN)
    var = sq_ref[...] * (1.0 / BN) - mean * mean
    scale = g_ref[...] * lax.rsqrt(var + 1e-5)
    shift = b_ref[...] - mean * scale
    out_ref[...] = jax.nn.gelu(or_ref[...] * scale + shift)


_tc_b = pl.pallas_call(
    _tc_bngelu,
    grid=(GRID_A,),
    in_specs=[
        pl.BlockSpec((RB, O), lambda i: (i, 0)),
        pl.BlockSpec((1, O), lambda i: (0, 0)),
        pl.BlockSpec((1, O), lambda i: (0, 0)),
        pl.BlockSpec((1, O), lambda i: (0, 0)),
        pl.BlockSpec((1, O), lambda i: (0, 0)),
    ],
    out_specs=pl.BlockSpec((RB, O), lambda i: (i, 0)),
    out_shape=jax.ShapeDtypeStruct((BN, O), jnp.float32),
)


def kernel(x, y, edge_index, W, gamma, beta):
    xt = x[:, :, :, 0].transpose(0, 2, 1).reshape(BN, C)
    yt = y[:, :, :, 0].transpose(0, 2, 1).reshape(BN, C)
    off = (jnp.arange(B, dtype=jnp.int32) * N).reshape(B, 1, 1)
    e0f = (edge_index[0] + off).reshape(-1)
    e1f = (edge_index[1] + off).reshape(-1)
    mr = _sc_call(xt, yt, e1f, e0f).reshape(BN, C)
    out_raw, s, q = _tc_a(xt, mr, W)
    out = _tc_b(out_raw, s, q, gamma.reshape(1, O), beta.reshape(1, O))
    return out.reshape(B, N, O).transpose(0, 2, 1)[..., None]
